# Initial kernel scaffold; baseline (speedup 1.0000x reference)
import jax
import jax.numpy as jnp
from jax.experimental import pallas as pl


def kernel(input, embed_weight):
    raise NotImplementedError("write your pallas kernel here")


# speedup vs baseline: 9.2495x; 9.2495x over previous
import jax
import jax.numpy as jnp
from jax import lax
from jax.experimental import pallas as pl
from jax.experimental.pallas import tpu as pltpu
from jax.experimental.pallas import tpu_sc as plsc

NUM_EMB = 100000
DIM = 128
NC = 2
NS = 16
NW = NC * NS
CH = 128
TOTAL = 4096 * 200
PER_W = TOTAL // NW
NCHUNK = PER_W // CH


def _body(idx_hbm, table_hbm, out_hbm, idx_v, rows0, rows1, sem0, sem1):
    wid = lax.axis_index("s") * NC + lax.axis_index("c")
    pltpu.sync_copy(idx_hbm.at[wid], idx_v)
    base = wid * PER_W

    pltpu.async_copy(table_hbm.at[idx_v.at[0]], rows0, sem0)
    pltpu.async_copy(table_hbm.at[idx_v.at[1]], rows1, sem1)

    bufs = ((rows0, sem0), (rows1, sem1))

    def step(i, carry):
        g = 2 * i
        for k, (rows, sem) in enumerate(bufs):
            ch = g + k
            pltpu.make_async_copy(table_hbm.at[idx_v.at[ch]], rows, sem).wait()
            pltpu.sync_copy(rows, out_hbm.at[pl.ds(base + ch * CH, CH)])
            pltpu.async_copy(table_hbm.at[idx_v.at[ch + 2]], rows, sem)
        return carry

    lax.fori_loop(0, NCHUNK // 2 - 1, step, 0)

    for k, (rows, sem) in enumerate(bufs):
        ch = NCHUNK - 2 + k
        pltpu.make_async_copy(table_hbm.at[idx_v.at[ch]], rows, sem).wait()
        pltpu.sync_copy(rows, out_hbm.at[pl.ds(base + ch * CH, CH)])


_sc_gather = pl.kernel(
    _body,
    out_type=jax.ShapeDtypeStruct((TOTAL, DIM), jnp.float32),
    mesh=plsc.VectorSubcoreMesh(core_axis_name="c", subcore_axis_name="s"),
    scratch_types=[
        pltpu.VMEM((NCHUNK, CH), jnp.int32),
        pltpu.VMEM((CH, DIM), jnp.float32),
        pltpu.VMEM((CH, DIM), jnp.float32),
        pltpu.SemaphoreType.DMA,
        pltpu.SemaphoreType.DMA,
    ],
)


def kernel(input, embed_weight):
    idx = input.reshape(NW, NCHUNK, CH).astype(jnp.int32)
    out = _sc_gather(idx, embed_weight)
    return out.reshape(input.shape[0], input.shape[1], DIM)
